# baseline (speedup 1.0000x reference)
import functools

import jax
import jax.numpy as jnp
from jax import lax
from jax.experimental import pallas as pl
from jax.experimental.pallas import tpu as pltpu
from jax.experimental.pallas import tpu_sc as plsc

N = 10000
E = 320000
D = 128
H = 256
G = 64
HH = H // 2

NC, NS = 2, 16
NW = NC * NS

CH = 125
EROWS = E // CH
ROWS_PER_SUB = EROWS // NS
ROWS_PER_W = EROWS // NW

NP = 10240
NPS = NP // NS
RB = 64
NRB = NPS // RB
SB = 16

_f32 = jnp.float32


def _sc_mesh():
    return plsc.VectorSubcoreMesh(
        core_axis_name="c", subcore_axis_name="s", num_cores=NC, num_subcores=NS
    )


DW = HH


def _fill_const(ref, rows, cols, value):
    def outer(i, carry):
        def inner(k, carry2):
            ref[i, pl.ds(k * 16, 16)] = jnp.full((16,), value, _f32)
            return carry2
        return lax.fori_loop(0, cols // 16, inner, carry)
    lax.fori_loop(0, rows, outer, 0)


def _deg_body(dst_hbm, degp_hbm, hist, idxv, ones, buf, ssem):
    c = lax.axis_index("c")
    s = lax.axis_index("s")
    wid = c * NS + s

    _fill_const(ones, CH, DW, 1.0)
    _fill_const(buf, RB, DW, 0.0)

    def z(k, carry):
        pltpu.sync_copy(buf, hist.at[pl.ds(s * NPS + k * RB, RB)])
        return carry

    lax.fori_loop(0, NRB, z, 0)
    pltpu.sync_copy(dst_hbm.at[pl.ds(wid * ROWS_PER_W, ROWS_PER_W)], idxv)
    plsc.subcore_barrier()

    def fire(j, carry):
        pltpu.async_copy(ones, hist.at[idxv.at[j]], ssem, add=True)
        return carry

    lax.fori_loop(0, ROWS_PER_W, fire, 0)

    def drain(j, carry):
        pltpu.make_async_copy(ones, hist.at[idxv.at[0]], ssem).wait()
        return carry

    lax.fori_loop(0, ROWS_PER_W, drain, 0)
    plsc.subcore_barrier()

    def rb_phase(core):
        def rb(k, carry):
            sl = pl.ds(s * NPS + k * RB, RB)
            pltpu.sync_copy(hist.at[sl], buf)
            pltpu.sync_copy(buf, degp_hbm.at[core].at[sl])
            return carry

        lax.fori_loop(0, NRB, rb, 0)

    @pl.when(c == 0)
    def _():
        rb_phase(0)

    @pl.when(c == 1)
    def _():
        rb_phase(1)


def _gs_body(pre_hbm, src_hbm, dst_hbm, acc_hbm, accsh,
             sidx0, didx0, sidx1, didx1, bufa, bufb,
             gsema, gsemb, ssema, ssemb, isem):
    c = lax.axis_index("c")
    s = lax.axis_index("s")

    nbase = s * NPS
    ebase = s * ROWS_PER_SUB
    NPAIR = ROWS_PER_SUB // 2
    NBLK = ROWS_PER_SUB // SB
    cinit = bufa.at[pl.ds(0, RB)]

    def init_phase(core):
        pre = pre_hbm.at[core]
        sl = pl.ds(nbase, NPS)
        pltpu.sync_copy(pre.at[sl], accsh.at[sl])

    def g_start(pre, bkpar, jrow, buf, sem):
        @pl.when(bkpar == 0)
        def _():
            pltpu.async_copy(pre.at[sidx0.at[jrow]], buf, sem)

        @pl.when(bkpar == 1)
        def _():
            pltpu.async_copy(pre.at[sidx1.at[jrow]], buf, sem)

    def g_wait(pre, buf, sem):
        pltpu.make_async_copy(pre.at[sidx0.at[0]], buf, sem).wait()

    def s_start(bkpar, jrow, buf, sem):
        @pl.when(bkpar == 0)
        def _():
            pltpu.async_copy(buf, accsh.at[didx0.at[jrow]], sem, add=True)

        @pl.when(bkpar == 1)
        def _():
            pltpu.async_copy(buf, accsh.at[didx1.at[jrow]], sem, add=True)

    def s_wait(buf, sem):
        pltpu.make_async_copy(buf, accsh.at[didx0.at[0]], sem).wait()

    def stage_start(bknext):
        bsl = pl.ds(ebase + bknext * SB, SB)

        @pl.when(bknext % 2 == 0)
        def _():
            pltpu.async_copy(src_hbm.at[bsl], sidx0, isem)
            pltpu.async_copy(dst_hbm.at[bsl], didx0, isem)

        @pl.when(bknext % 2 == 1)
        def _():
            pltpu.async_copy(src_hbm.at[bsl], sidx1, isem)
            pltpu.async_copy(dst_hbm.at[bsl], didx1, isem)

    def stage_wait():
        pltpu.make_async_copy(src_hbm.at[pl.ds(0, SB)], sidx0, isem).wait()
        pltpu.make_async_copy(dst_hbm.at[pl.ds(0, SB)], didx0, isem).wait()

    def main_phase(core):
        pre = pre_hbm.at[core]
        pltpu.sync_copy(src_hbm.at[pl.ds(ebase, SB)], sidx0)
        pltpu.sync_copy(dst_hbm.at[pl.ds(ebase, SB)], didx0)
        g_start(pre, 0, 0, bufa, gsema)

        def body(p, carry):
            j0 = 2 * p
            bk = j0 // SB
            bkpar = bk % 2
            r0 = j0 - bk * SB
            g_wait(pre, bufa, gsema)

            @pl.when(p > 0)
            def _():
                s_wait(bufb, ssemb)

            s_start(bkpar, r0, bufa, ssema)
            g_start(pre, bkpar, r0 + 1, bufb, gsemb)

            @pl.when(jnp.logical_and(p % 8 == 1, bk < NBLK - 1))
            def _():
                stage_start(bk + 1)

            g_wait(pre, bufb, gsemb)
            s_wait(bufa, ssema)
            s_start(bkpar, r0 + 1, bufb, ssemb)

            @pl.when(jnp.logical_and(p % 8 == 7, p < NPAIR - 8))
            def _():
                stage_wait()

            @pl.when(p < NPAIR - 1)
            def _():
                j2 = j0 + 2
                g_start(pre, (j2 // SB) % 2, j2 % SB, bufa, gsema)

            return carry

        lax.fori_loop(0, NPAIR, body, 0)
        s_wait(bufb, ssemb)

    def rb_phase(core):
        out = acc_hbm.at[core]
        sl = pl.ds(nbase, NPS)
        pltpu.sync_copy(accsh.at[sl], out.at[sl])

    @pl.when(c == 0)
    def _():
        init_phase(0)

    @pl.when(c == 1)
    def _():
        init_phase(1)

    plsc.subcore_barrier()

    @pl.when(c == 0)
    def _():
        main_phase(0)

    @pl.when(c == 1)
    def _():
        main_phase(1)

    plsc.subcore_barrier()

    @pl.when(c == 0)
    def _():
        rb_phase(0)

    @pl.when(c == 1)
    def _():
        rb_phase(1)


@functools.lru_cache(maxsize=None)
def _deg_kernel():
    return pl.kernel(
        _deg_body,
        out_type=jax.ShapeDtypeStruct((NC, NP, DW), _f32),
        mesh=_sc_mesh(),
        scratch_types=[
            pltpu.VMEM_SHARED((NP, DW), _f32),
            pltpu.VMEM((ROWS_PER_W, CH), jnp.int32),
            pltpu.VMEM((CH, DW), _f32),
            pltpu.VMEM((RB, DW), _f32),
            pltpu.SemaphoreType.DMA,
        ],
    )


@functools.lru_cache(maxsize=None)
def _gs_kernel():
    return pl.kernel(
        _gs_body,
        out_type=jax.ShapeDtypeStruct((NC, NP, HH), _f32),
        mesh=_sc_mesh(),
        scratch_types=[
            pltpu.VMEM_SHARED((NP, HH), _f32),
            pltpu.VMEM((SB, CH), jnp.int32),
            pltpu.VMEM((SB, CH), jnp.int32),
            pltpu.VMEM((SB, CH), jnp.int32),
            pltpu.VMEM((SB, CH), jnp.int32),
            pltpu.VMEM((CH, HH), _f32),
            pltpu.VMEM((CH, HH), _f32),
            pltpu.SemaphoreType.DMA,
            pltpu.SemaphoreType.DMA,
            pltpu.SemaphoreType.DMA,
            pltpu.SemaphoreType.DMA,
            pltpu.SemaphoreType.DMA,
        ],
    )


MB = NP // 5


def _tc_deg_body(degp_ref, dsq_ref):
    deg = 1.0 + degp_ref[0, :, 0:1] + degp_ref[1, :, 0:1]
    dsq_ref[...] = lax.rsqrt(deg)


def _mm_body(h_ref, w_ref, dsq_ref, pre_ref):
    m = jnp.dot(h_ref[...], w_ref[...], preferred_element_type=_f32,
                precision=lax.Precision.HIGHEST) * dsq_ref[...]
    pre_ref[0] = m[:, :HH]
    pre_ref[1] = m[:, HH:]


def _bn_act(acc_ref, dsq_ref, b_ref, g_ref, bt_ref):
    conv = (
        jnp.concatenate([acc_ref[0, :N, :], acc_ref[1, :N, :]], axis=-1)
        * dsq_ref[:N, :]
        + b_ref[...]
    )
    mu = jnp.mean(conv, axis=0, keepdims=True)
    var = jnp.mean((conv - mu) ** 2, axis=0, keepdims=True)
    t = (conv - mu) * lax.rsqrt(var + 1e-5) * g_ref[...] + bt_ref[...]
    return jnp.maximum(t, 0.0)


def _make_bn_body(has_res):
    if has_res:
        def body(acc_ref, dsq_ref, b_ref, g_ref, bt_ref, res_ref, h_ref):
            h = _bn_act(acc_ref, dsq_ref, b_ref, g_ref, bt_ref) + res_ref[:N, :]
            h_ref[:N, :] = h
            h_ref[N:, :] = jnp.zeros((NP - N, H), _f32)
    else:
        def body(acc_ref, dsq_ref, b_ref, g_ref, bt_ref, h_ref):
            h = _bn_act(acc_ref, dsq_ref, b_ref, g_ref, bt_ref)
            h_ref[:N, :] = h
            h_ref[N:, :] = jnp.zeros((NP - N, H), _f32)
    return body


def _fin_body(acc_ref, dsq_ref, b_ref, g_ref, bt_ref, res_ref, batch_ref,
              linw_ref, linb_ref, out_ref):
    h = _bn_act(acc_ref, dsq_ref, b_ref, g_ref, bt_ref) + res_ref[:N, :]
    y = jnp.dot(h, linw_ref[...], preferred_element_type=_f32)
    yc = jnp.concatenate([y, jnp.ones_like(y)], axis=1)
    onehot = (
        lax.broadcasted_iota(jnp.int32, (G, N), 0) == batch_ref[...]
    ).astype(_f32)
    s2 = jnp.dot(onehot, yc, preferred_element_type=_f32)
    out_ref[...] = s2[:, 0:1] / jnp.maximum(s2[:, 1:2], 1.0) + linb_ref[...]


_tc_params = pltpu.CompilerParams(vmem_limit_bytes=63 * 1024 * 1024)

_tc_deg = pl.pallas_call(
    _tc_deg_body,
    out_shape=jax.ShapeDtypeStruct((NP, 1), _f32),
    compiler_params=_tc_params,
)


def _mk_mm(din):
    return pl.pallas_call(
        _mm_body,
        grid=(NP // MB,),
        in_specs=[
            pl.BlockSpec((MB, din), lambda i: (i, 0)),
            pl.BlockSpec((din, H), lambda i: (0, 0)),
            pl.BlockSpec((MB, 1), lambda i: (i, 0)),
        ],
        out_specs=pl.BlockSpec((NC, MB, HH), lambda i: (0, i, 0)),
        out_shape=jax.ShapeDtypeStruct((NC, NP, HH), _f32),
        compiler_params=_tc_params,
    )


_mm_d = _mk_mm(D)
_mm_h = _mk_mm(H)

_tc_bn0 = pl.pallas_call(
    _make_bn_body(False),
    out_shape=jax.ShapeDtypeStruct((NP, H), _f32),
    compiler_params=_tc_params,
)

_tc_bn1 = pl.pallas_call(
    _make_bn_body(True),
    out_shape=jax.ShapeDtypeStruct((NP, H), _f32),
    compiler_params=_tc_params,
)

_tc_fin = pl.pallas_call(
    _fin_body,
    out_shape=jax.ShapeDtypeStruct((G, 1), _f32),
    compiler_params=_tc_params,
)


def kernel(x, edge_index, batch, W0, b0, W1, b1, W2, b2,
           g0, bt0, g1, bt1, g2, bt2, linW, linb):
    src2d = edge_index[0].reshape(EROWS, CH)
    dst2d = edge_index[1].reshape(EROWS, CH)
    batch_row = batch.reshape(1, N)
    b0r, b1r, b2r = b0.reshape(1, H), b1.reshape(1, H), b2.reshape(1, H)
    g0r, g1r, g2r = g0.reshape(1, H), g1.reshape(1, H), g2.reshape(1, H)
    t0r, t1r, t2r = bt0.reshape(1, H), bt1.reshape(1, H), bt2.reshape(1, H)
    linbr = linb.reshape(1, 1)
    xp = jnp.concatenate([x, jnp.zeros((NP - N, D), _f32)], axis=0)

    deg_k = _deg_kernel()
    gs_k = _gs_kernel()
    degp = deg_k(dst2d)
    dsq = _tc_deg(degp)
    pre0 = _mm_d(xp, W0, dsq)
    acc0 = gs_k(pre0, src2d, dst2d)
    ha = _tc_bn0(acc0, dsq, b0r, g0r, t0r)
    pre1 = _mm_h(ha, W1, dsq)
    acc1 = gs_k(pre1, src2d, dst2d)
    hb = _tc_bn1(acc1, dsq, b1r, g1r, t1r, ha)
    pre2 = _mm_h(hb, W2, dsq)
    acc2 = gs_k(pre2, src2d, dst2d)
    out = _tc_fin(acc2, dsq, b2r, g2r, t2r, hb, batch_row, linW, linbr)
    return out


# speedup vs baseline: 1.0368x; 1.0368x over previous
import functools

import jax
import jax.numpy as jnp
from jax import lax
from jax.experimental import pallas as pl
from jax.experimental.pallas import tpu as pltpu
from jax.experimental.pallas import tpu_sc as plsc

N = 10000
E = 320000
D = 128
H = 256
G = 64
HH = H // 2

NC, NS = 2, 16
NW = NC * NS

CH = 125
EROWS = E // CH
ROWS_PER_SUB = EROWS // NS
ROWS_PER_W = EROWS // NW

NP = 10240
NPS = NP // NS
RB = 64
NRB = NPS // RB
SB = 16

_f32 = jnp.float32


def _sc_mesh():
    return plsc.VectorSubcoreMesh(
        core_axis_name="c", subcore_axis_name="s", num_cores=NC, num_subcores=NS
    )


DW = HH


def _fill_const(ref, rows, cols, value):
    def outer(i, carry):
        def inner(k, carry2):
            ref[i, pl.ds(k * 16, 16)] = jnp.full((16,), value, _f32)
            return carry2
        return lax.fori_loop(0, cols // 16, inner, carry)
    lax.fori_loop(0, rows, outer, 0)


def _deg_body(dst_hbm, degp_hbm, hist, idxv, ones, buf, ssem):
    c = lax.axis_index("c")
    s = lax.axis_index("s")
    wid = c * NS + s

    _fill_const(ones, CH, DW, 1.0)
    _fill_const(buf, RB, DW, 0.0)

    def z(k, carry):
        pltpu.sync_copy(buf, hist.at[pl.ds(s * NPS + k * RB, RB)])
        return carry

    lax.fori_loop(0, NRB, z, 0)
    pltpu.sync_copy(dst_hbm.at[pl.ds(wid * ROWS_PER_W, ROWS_PER_W)], idxv)
    plsc.subcore_barrier()

    def fire(j, carry):
        pltpu.async_copy(ones, hist.at[idxv.at[j]], ssem, add=True)
        return carry

    lax.fori_loop(0, ROWS_PER_W, fire, 0)

    def drain(j, carry):
        pltpu.make_async_copy(ones, hist.at[idxv.at[0]], ssem).wait()
        return carry

    lax.fori_loop(0, ROWS_PER_W, drain, 0)
    plsc.subcore_barrier()

    def rb_phase(core):
        def rb(k, carry):
            sl = pl.ds(s * NPS + k * RB, RB)
            pltpu.sync_copy(hist.at[sl], buf)
            pltpu.sync_copy(buf, degp_hbm.at[core].at[sl])
            return carry

        lax.fori_loop(0, NRB, rb, 0)

    @pl.when(c == 0)
    def _():
        rb_phase(0)

    @pl.when(c == 1)
    def _():
        rb_phase(1)


def _gs_body(pre_hbm, src_hbm, dst_hbm, acc_hbm, accsh,
             sidx0, didx0, sidx1, didx1, bufa, bufb,
             gsema, gsemb, ssema, ssemb, isem):
    c = lax.axis_index("c")
    s = lax.axis_index("s")

    nbase = s * NPS
    ebase = s * ROWS_PER_SUB
    NPAIR = ROWS_PER_SUB // 2
    NBLK = ROWS_PER_SUB // SB
    cinit = bufa.at[pl.ds(0, RB)]

    def init_phase(core):
        pre = pre_hbm.at[core]
        sl = pl.ds(nbase, NPS)
        pltpu.sync_copy(pre.at[sl], accsh.at[sl])

    def g_start(pre, bkpar, jrow, buf, sem):
        @pl.when(bkpar == 0)
        def _():
            pltpu.async_copy(pre.at[sidx0.at[jrow]], buf, sem)

        @pl.when(bkpar == 1)
        def _():
            pltpu.async_copy(pre.at[sidx1.at[jrow]], buf, sem)

    def g_wait(pre, buf, sem):
        pltpu.make_async_copy(pre.at[sidx0.at[0]], buf, sem).wait()

    def s_start(bkpar, jrow, buf, sem):
        @pl.when(bkpar == 0)
        def _():
            pltpu.async_copy(buf, accsh.at[didx0.at[jrow]], sem, add=True)

        @pl.when(bkpar == 1)
        def _():
            pltpu.async_copy(buf, accsh.at[didx1.at[jrow]], sem, add=True)

    def s_wait(buf, sem):
        pltpu.make_async_copy(buf, accsh.at[didx0.at[0]], sem).wait()

    def stage_start(bknext):
        bsl = pl.ds(ebase + bknext * SB, SB)

        @pl.when(bknext % 2 == 0)
        def _():
            pltpu.async_copy(src_hbm.at[bsl], sidx0, isem)
            pltpu.async_copy(dst_hbm.at[bsl], didx0, isem)

        @pl.when(bknext % 2 == 1)
        def _():
            pltpu.async_copy(src_hbm.at[bsl], sidx1, isem)
            pltpu.async_copy(dst_hbm.at[bsl], didx1, isem)

    def stage_wait():
        pltpu.make_async_copy(src_hbm.at[pl.ds(0, SB)], sidx0, isem).wait()
        pltpu.make_async_copy(dst_hbm.at[pl.ds(0, SB)], didx0, isem).wait()

    def main_phase(core):
        pre = pre_hbm.at[core]
        pltpu.sync_copy(src_hbm.at[pl.ds(ebase, SB)], sidx0)
        pltpu.sync_copy(dst_hbm.at[pl.ds(ebase, SB)], didx0)
        g_start(pre, 0, 0, bufa, gsema)

        def body(p, carry):
            j0 = 2 * p
            bk = j0 // SB
            bkpar = bk % 2
            r0 = j0 - bk * SB
            g_wait(pre, bufa, gsema)

            @pl.when(p > 0)
            def _():
                s_wait(bufb, ssemb)

            s_start(bkpar, r0, bufa, ssema)
            g_start(pre, bkpar, r0 + 1, bufb, gsemb)

            @pl.when(jnp.logical_and(p % 8 == 1, bk < NBLK - 1))
            def _():
                stage_start(bk + 1)

            g_wait(pre, bufb, gsemb)
            s_wait(bufa, ssema)
            s_start(bkpar, r0 + 1, bufb, ssemb)

            @pl.when(jnp.logical_and(p % 8 == 7, p < NPAIR - 8))
            def _():
                stage_wait()

            @pl.when(p < NPAIR - 1)
            def _():
                j2 = j0 + 2
                g_start(pre, (j2 // SB) % 2, j2 % SB, bufa, gsema)

            return carry

        lax.fori_loop(0, NPAIR, body, 0)
        s_wait(bufb, ssemb)

    def rb_phase(core):
        out = acc_hbm.at[core]
        sl = pl.ds(nbase, NPS)
        pltpu.sync_copy(accsh.at[sl], out.at[sl])

    @pl.when(c == 0)
    def _():
        init_phase(0)

    @pl.when(c == 1)
    def _():
        init_phase(1)

    plsc.subcore_barrier()

    @pl.when(c == 0)
    def _():
        main_phase(0)

    @pl.when(c == 1)
    def _():
        main_phase(1)

    plsc.subcore_barrier()

    @pl.when(c == 0)
    def _():
        rb_phase(0)

    @pl.when(c == 1)
    def _():
        rb_phase(1)


@functools.lru_cache(maxsize=None)
def _deg_kernel():
    return pl.kernel(
        _deg_body,
        out_type=jax.ShapeDtypeStruct((NC, NP, DW), _f32),
        mesh=_sc_mesh(),
        scratch_types=[
            pltpu.VMEM_SHARED((NP, DW), _f32),
            pltpu.VMEM((ROWS_PER_W, CH), jnp.int32),
            pltpu.VMEM((CH, DW), _f32),
            pltpu.VMEM((RB, DW), _f32),
            pltpu.SemaphoreType.DMA,
        ],
    )


@functools.lru_cache(maxsize=None)
def _gs_kernel():
    return pl.kernel(
        _gs_body,
        out_type=jax.ShapeDtypeStruct((NC, NP, HH), _f32),
        mesh=_sc_mesh(),
        scratch_types=[
            pltpu.VMEM_SHARED((NP, HH), _f32),
            pltpu.VMEM((SB, CH), jnp.int32),
            pltpu.VMEM((SB, CH), jnp.int32),
            pltpu.VMEM((SB, CH), jnp.int32),
            pltpu.VMEM((SB, CH), jnp.int32),
            pltpu.VMEM((CH, HH), _f32),
            pltpu.VMEM((CH, HH), _f32),
            pltpu.SemaphoreType.DMA,
            pltpu.SemaphoreType.DMA,
            pltpu.SemaphoreType.DMA,
            pltpu.SemaphoreType.DMA,
            pltpu.SemaphoreType.DMA,
        ],
    )


def _tc0_body(x_ref, w_ref, degp_ref, pre_ref, dsq_ref):
    deg = 1.0 + degp_ref[0, :N, 0:1] + degp_ref[1, :N, 0:1]
    dsq = lax.rsqrt(deg)
    dsq_ref[...] = dsq
    m = jnp.dot(x_ref[...], w_ref[...], preferred_element_type=_f32) * dsq
    pre_ref[0, :N, :] = m[:, :HH]
    pre_ref[1, :N, :] = m[:, HH:]
    pre_ref[0, N:, :] = jnp.zeros((NP - N, HH), _f32)
    pre_ref[1, N:, :] = jnp.zeros((NP - N, HH), _f32)


def _bn_act(acc_ref, dsq_ref, b_ref, g_ref, bt_ref):
    conv = (
        jnp.concatenate([acc_ref[0, :N, :], acc_ref[1, :N, :]], axis=-1)
        * dsq_ref[...]
        + b_ref[...]
    )
    mu = jnp.mean(conv, axis=0, keepdims=True)
    var = jnp.mean((conv - mu) ** 2, axis=0, keepdims=True)
    t = (conv - mu) * lax.rsqrt(var + 1e-5) * g_ref[...] + bt_ref[...]
    return jnp.maximum(t, 0.0)


def _make_mid_body(has_res):
    if has_res:
        def body(acc_ref, dsq_ref, b_ref, g_ref, bt_ref, res_ref, w_ref,
                 h_ref, pre_ref):
            h = _bn_act(acc_ref, dsq_ref, b_ref, g_ref, bt_ref) + res_ref[...]
            h_ref[...] = h
            pre = jnp.dot(h, w_ref[...], preferred_element_type=_f32) * dsq_ref[...]
            pre_ref[0, :N, :] = pre[:, :HH]
            pre_ref[1, :N, :] = pre[:, HH:]
            pre_ref[0, N:, :] = jnp.zeros((NP - N, HH), _f32)
            pre_ref[1, N:, :] = jnp.zeros((NP - N, HH), _f32)
    else:
        def body(acc_ref, dsq_ref, b_ref, g_ref, bt_ref, w_ref, h_ref, pre_ref):
            h = _bn_act(acc_ref, dsq_ref, b_ref, g_ref, bt_ref)
            h_ref[...] = h
            pre = jnp.dot(h, w_ref[...], preferred_element_type=_f32) * dsq_ref[...]
            pre_ref[0, :N, :] = pre[:, :HH]
            pre_ref[1, :N, :] = pre[:, HH:]
            pre_ref[0, N:, :] = jnp.zeros((NP - N, HH), _f32)
            pre_ref[1, N:, :] = jnp.zeros((NP - N, HH), _f32)
    return body


def _fin_body(acc_ref, dsq_ref, b_ref, g_ref, bt_ref, res_ref, batch_ref,
              linw_ref, linb_ref, out_ref):
    h = _bn_act(acc_ref, dsq_ref, b_ref, g_ref, bt_ref) + res_ref[...]
    y = jnp.dot(h, linw_ref[...], preferred_element_type=_f32)
    yc = jnp.concatenate([y, jnp.ones_like(y)], axis=1)
    onehot = (
        lax.broadcasted_iota(jnp.int32, (G, N), 0) == batch_ref[...]
    ).astype(_f32)
    s2 = jnp.dot(onehot, yc, preferred_element_type=_f32)
    out_ref[...] = s2[:, 0:1] / jnp.maximum(s2[:, 1:2], 1.0) + linb_ref[...]


_tc_params = pltpu.CompilerParams(vmem_limit_bytes=63 * 1024 * 1024)

_tc0 = pl.pallas_call(
    _tc0_body,
    out_shape=[
        jax.ShapeDtypeStruct((NC, NP, HH), _f32),
        jax.ShapeDtypeStruct((N, 1), _f32),
    ],
    compiler_params=_tc_params,
)

_tc_mid0 = pl.pallas_call(
    _make_mid_body(False),
    out_shape=[
        jax.ShapeDtypeStruct((N, H), _f32),
        jax.ShapeDtypeStruct((NC, NP, HH), _f32),
    ],
    compiler_params=_tc_params,
)

_tc_mid1 = pl.pallas_call(
    _make_mid_body(True),
    out_shape=[
        jax.ShapeDtypeStruct((N, H), _f32),
        jax.ShapeDtypeStruct((NC, NP, HH), _f32),
    ],
    compiler_params=_tc_params,
)

_tc_fin = pl.pallas_call(
    _fin_body,
    out_shape=jax.ShapeDtypeStruct((G, 1), _f32),
    compiler_params=_tc_params,
)


def kernel(x, edge_index, batch, W0, b0, W1, b1, W2, b2,
           g0, bt0, g1, bt1, g2, bt2, linW, linb):
    src2d = edge_index[0].reshape(EROWS, CH)
    dst2d = edge_index[1].reshape(EROWS, CH)
    batch_row = batch.reshape(1, N)
    b0r, b1r, b2r = b0.reshape(1, H), b1.reshape(1, H), b2.reshape(1, H)
    g0r, g1r, g2r = g0.reshape(1, H), g1.reshape(1, H), g2.reshape(1, H)
    t0r, t1r, t2r = bt0.reshape(1, H), bt1.reshape(1, H), bt2.reshape(1, H)
    linbr = linb.reshape(1, 1)

    deg_k = _deg_kernel()
    gs_k = _gs_kernel()
    degp = deg_k(dst2d)
    pre0, dsq = _tc0(x, W0, degp)
    acc0 = gs_k(pre0, src2d, dst2d)
    ha, pre1 = _tc_mid0(acc0, dsq, b0r, g0r, t0r, W1)
    acc1 = gs_k(pre1, src2d, dst2d)
    hb, pre2 = _tc_mid1(acc1, dsq, b1r, g1r, t1r, ha, W2)
    acc2 = gs_k(pre2, src2d, dst2d)
    out = _tc_fin(acc2, dsq, b2r, g2r, t2r, hb, batch_row, linW, linbr)
    return out
